# Initial kernel scaffold; baseline (speedup 1.0000x reference)
import jax
import jax.numpy as jnp
from jax.experimental import pallas as pl


def kernel(x, edge_index, W0, b0, W1, b1):
    raise NotImplementedError("write your pallas kernel here")


# speedup vs baseline: 100.6427x; 100.6427x over previous
import functools

import jax
import jax.numpy as jnp
from jax import lax
from jax.experimental import pallas as pl
from jax.experimental.pallas import tpu as pltpu
from jax.experimental.pallas import tpu_sc as plsc

N_NODES = 10000
D_FEAT = 128
N_EDGES = 320000

NS = 16
LANES = 16
ROWS = 157
E_TILE = ROWS * 128
E_PAD = NS * E_TILE
NP = 10240
STRIPE = NP // NS


def _edge_body(src_hbm, dst_hbm, coef_hbm,
               src_v, dst_v, vals_v, g_v, stripe_v, dinv_v,
               cnt_sh, dinv_sh, t_sh, sem):
    s = lax.axis_index("s")
    ebase = s * E_TILE
    nbase = s * STRIPE

    pltpu.sync_copy(src_hbm.at[pl.ds(ebase, E_TILE)], src_v)
    pltpu.sync_copy(dst_hbm.at[pl.ds(ebase, E_TILE)], dst_v)

    def zero_body(i, _):
        stripe_v[pl.ds(i * LANES, LANES)] = jnp.zeros((LANES,), jnp.float32)
        return _
    lax.fori_loop(0, STRIPE // LANES, zero_body, None)
    pltpu.sync_copy(stripe_v, cnt_sh.at[pl.ds(nbase, STRIPE)])
    pltpu.sync_copy(stripe_v, t_sh.at[pl.ds(nbase, STRIPE)])
    plsc.subcore_barrier()

    def mask_body(i, _):
        sv = src_v[pl.ds(i * LANES, LANES)]
        dv = dst_v[pl.ds(i * LANES, LANES)]
        vals_v[pl.ds(i * LANES, LANES)] = jnp.where(
            sv != dv, jnp.float32(1.0), jnp.float32(0.0))
        return _
    lax.fori_loop(0, E_TILE // LANES, mask_body, None)
    pltpu.sync_copy(vals_v, cnt_sh.at[dst_v], add=True)
    plsc.subcore_barrier()

    pltpu.sync_copy(cnt_sh.at[pl.ds(nbase, STRIPE)], stripe_v)

    def dinv_body(i, _):
        c = stripe_v[pl.ds(i * LANES, LANES)]
        xdeg = c + jnp.float32(1.0)
        ii = lax.bitcast_convert_type(xdeg, jnp.int32)
        ii = jnp.int32(0x5F3759DF) - (ii >> 1)
        y = lax.bitcast_convert_type(ii, jnp.float32)
        for _unused in range(3):
            y = y * (jnp.float32(1.5) - jnp.float32(0.5) * xdeg * y * y)
        dinv_v[pl.ds(i * LANES, LANES)] = y
        return _
    lax.fori_loop(0, STRIPE // LANES, dinv_body, None)
    pltpu.sync_copy(dinv_v, dinv_sh.at[pl.ds(nbase, STRIPE)])
    plsc.subcore_barrier()

    pltpu.async_copy(dinv_sh.at[dst_v], g_v, sem).wait()

    def prod_body(i, _):
        g_v[pl.ds(i * LANES, LANES)] = (
            g_v[pl.ds(i * LANES, LANES)] * vals_v[pl.ds(i * LANES, LANES)])
        return _
    lax.fori_loop(0, E_TILE // LANES, prod_body, None)
    pltpu.sync_copy(g_v, t_sh.at[src_v], add=True)
    plsc.subcore_barrier()

    pltpu.sync_copy(t_sh.at[pl.ds(nbase, STRIPE)], stripe_v)

    def coef_body(i, _):
        dv = dinv_v[pl.ds(i * LANES, LANES)]
        tv = stripe_v[pl.ds(i * LANES, LANES)]
        stripe_v[pl.ds(i * LANES, LANES)] = dv * (tv + dv)
        return _
    lax.fori_loop(0, STRIPE // LANES, coef_body, None)
    pltpu.sync_copy(stripe_v, coef_hbm.at[pl.ds(nbase, STRIPE)])


_edge_kernel = pl.kernel(
    _edge_body,
    out_type=jax.ShapeDtypeStruct((NP,), jnp.float32),
    mesh=plsc.VectorSubcoreMesh(
        core_axis_name="c", subcore_axis_name="s", num_cores=1),
    scratch_types=[
        pltpu.VMEM((E_TILE,), jnp.int32),
        pltpu.VMEM((E_TILE,), jnp.int32),
        pltpu.VMEM((E_TILE,), jnp.float32),
        pltpu.VMEM((E_TILE,), jnp.float32),
        pltpu.VMEM((STRIPE,), jnp.float32),
        pltpu.VMEM((STRIPE,), jnp.float32),
        pltpu.VMEM_SHARED((NP,), jnp.float32),
        pltpu.VMEM_SHARED((NP,), jnp.float32),
        pltpu.VMEM_SHARED((NP,), jnp.float32),
        pltpu.SemaphoreType.DMA,
    ],
)


def _dense_body(x_ref, w0_ref, b0_ref, w1_ref, b1_ref, coef_ref, out_ref):
    coef = coef_ref[...]
    h = lax.dot_general(
        x_ref[...], w0_ref[...], (((1,), (1,)), ((), ())),
        preferred_element_type=jnp.float32)
    h = h * coef + b0_ref[...]
    mean = jnp.mean(h, axis=0, keepdims=True)
    cen = h - mean
    var = jnp.mean(cen * cen, axis=0, keepdims=True)
    hn = cen * lax.rsqrt(var + jnp.float32(1e-5))
    hr = jnp.maximum(hn, jnp.float32(0.0))
    h2 = lax.dot_general(
        hr, w1_ref[...], (((1,), (1,)), ((), ())),
        preferred_element_type=jnp.float32)
    out_ref[...] = h2 * coef + b1_ref[...]


@functools.partial(jax.jit, static_argnames=())
def kernel(x, edge_index, W0, b0, W1, b1):
    src = edge_index[0]
    dst = edge_index[1]
    pad = jnp.zeros((E_PAD - N_EDGES,), jnp.int32)
    srcp = jnp.concatenate([src, pad])
    dstp = jnp.concatenate([dst, pad])

    coef_full = _edge_kernel(srcp, dstp)
    coef_col = coef_full[:N_NODES].reshape(N_NODES, 1)

    out = pl.pallas_call(
        _dense_body,
        out_shape=jax.ShapeDtypeStruct((N_NODES, D_FEAT), jnp.float32),
    )(x, W0, b0.reshape(1, D_FEAT), W1, b1.reshape(1, D_FEAT), coef_col)
    return out


# speedup vs baseline: 101.4689x; 1.0082x over previous
import functools

import jax
import jax.numpy as jnp
from jax import lax
from jax.experimental import pallas as pl
from jax.experimental.pallas import tpu as pltpu
from jax.experimental.pallas import tpu_sc as plsc

N_NODES = 10000
D_FEAT = 128
N_EDGES = 320000

NS = 16
LANES = 16
E_TILE = N_EDGES // NS
NP = 10240
STRIPE = NP // NS
U = 10


def _edge_body(src_hbm, dst_hbm, coef_hbm,
               src_v, dst_v, vals_v, g_v, stripe_v, dinv_v,
               cnt_sh, dinv_sh, t_sh, sem):
    s = lax.axis_index("s")
    ebase = s * E_TILE
    nbase = s * STRIPE

    pltpu.sync_copy(src_hbm.at[pl.ds(ebase, E_TILE)], src_v)
    pltpu.sync_copy(dst_hbm.at[pl.ds(ebase, E_TILE)], dst_v)

    def zero_body(i, _):
        for u in range(8):
            stripe_v[pl.ds(i * 8 * LANES + u * LANES, LANES)] = (
                jnp.zeros((LANES,), jnp.float32))
        return _
    lax.fori_loop(0, STRIPE // (8 * LANES), zero_body, None)
    pltpu.sync_copy(stripe_v, cnt_sh.at[pl.ds(nbase, STRIPE)])
    pltpu.sync_copy(stripe_v, t_sh.at[pl.ds(nbase, STRIPE)])
    plsc.subcore_barrier()

    def mask_body(i, _):
        for u in range(U):
            off = i * U * LANES + u * LANES
            sv = src_v[pl.ds(off, LANES)]
            dv = dst_v[pl.ds(off, LANES)]
            vals_v[pl.ds(off, LANES)] = jnp.where(
                sv != dv, jnp.float32(1.0), jnp.float32(0.0))
        return _
    lax.fori_loop(0, E_TILE // (U * LANES), mask_body, None)
    pltpu.sync_copy(vals_v, cnt_sh.at[dst_v], add=True)
    plsc.subcore_barrier()

    pltpu.sync_copy(cnt_sh.at[pl.ds(nbase, STRIPE)], stripe_v)

    def dinv_body(i, _):
        for u in range(8):
            off = i * 8 * LANES + u * LANES
            c = stripe_v[pl.ds(off, LANES)]
            xdeg = c + jnp.float32(1.0)
            ii = lax.bitcast_convert_type(xdeg, jnp.int32)
            ii = jnp.int32(0x5F3759DF) - (ii >> 1)
            y = lax.bitcast_convert_type(ii, jnp.float32)
            for _unused in range(3):
                y = y * (jnp.float32(1.5) - jnp.float32(0.5) * xdeg * y * y)
            dinv_v[pl.ds(off, LANES)] = y
        return _
    lax.fori_loop(0, STRIPE // (8 * LANES), dinv_body, None)
    pltpu.sync_copy(dinv_v, dinv_sh.at[pl.ds(nbase, STRIPE)])
    plsc.subcore_barrier()

    pltpu.async_copy(dinv_sh.at[dst_v], g_v, sem).wait()

    def prod_body(i, _):
        for u in range(U):
            off = i * U * LANES + u * LANES
            g_v[pl.ds(off, LANES)] = (
                g_v[pl.ds(off, LANES)] * vals_v[pl.ds(off, LANES)])
        return _
    lax.fori_loop(0, E_TILE // (U * LANES), prod_body, None)
    pltpu.sync_copy(g_v, t_sh.at[src_v], add=True)
    plsc.subcore_barrier()

    pltpu.sync_copy(t_sh.at[pl.ds(nbase, STRIPE)], stripe_v)

    def coef_body(i, _):
        for u in range(8):
            off = i * 8 * LANES + u * LANES
            dv = dinv_v[pl.ds(off, LANES)]
            tv = stripe_v[pl.ds(off, LANES)]
            stripe_v[pl.ds(off, LANES)] = dv * (tv + dv)
        return _
    lax.fori_loop(0, STRIPE // (8 * LANES), coef_body, None)
    pltpu.sync_copy(stripe_v, coef_hbm.at[pl.ds(nbase, STRIPE)])


_edge_kernel = pl.kernel(
    _edge_body,
    out_type=jax.ShapeDtypeStruct((NP,), jnp.float32),
    mesh=plsc.VectorSubcoreMesh(
        core_axis_name="c", subcore_axis_name="s", num_cores=1),
    scratch_types=[
        pltpu.VMEM((E_TILE,), jnp.int32),
        pltpu.VMEM((E_TILE,), jnp.int32),
        pltpu.VMEM((E_TILE,), jnp.float32),
        pltpu.VMEM((E_TILE,), jnp.float32),
        pltpu.VMEM((STRIPE,), jnp.float32),
        pltpu.VMEM((STRIPE,), jnp.float32),
        pltpu.VMEM_SHARED((NP,), jnp.float32),
        pltpu.VMEM_SHARED((NP,), jnp.float32),
        pltpu.VMEM_SHARED((NP,), jnp.float32),
        pltpu.SemaphoreType.DMA,
    ],
)


def _mm0_body(x_ref, w0_ref, h0_ref):
    h0_ref[...] = lax.dot_general(
        x_ref[...], w0_ref[...], (((1,), (1,)), ((), ())),
        preferred_element_type=jnp.float32)


def _dense_body(h0_ref, b0_ref, w1_ref, b1_ref, coef_ref, out_ref):
    coef = coef_ref[...]
    h = h0_ref[...] * coef + b0_ref[...]
    mean = jnp.mean(h, axis=0, keepdims=True)
    cen = h - mean
    var = jnp.mean(cen * cen, axis=0, keepdims=True)
    hn = cen * lax.rsqrt(var + jnp.float32(1e-5))
    hr = jnp.maximum(hn, jnp.float32(0.0))
    h2 = lax.dot_general(
        hr, w1_ref[...], (((1,), (1,)), ((), ())),
        preferred_element_type=jnp.float32)
    out_ref[...] = h2 * coef + b1_ref[...]


@functools.partial(jax.jit, static_argnames=())
def kernel(x, edge_index, W0, b0, W1, b1):
    coef_full = _edge_kernel(edge_index[0], edge_index[1])
    coef_col = coef_full[:N_NODES].reshape(N_NODES, 1)

    h0 = pl.pallas_call(
        _mm0_body,
        out_shape=jax.ShapeDtypeStruct((N_NODES, D_FEAT), jnp.float32),
    )(x, W0)

    out = pl.pallas_call(
        _dense_body,
        out_shape=jax.ShapeDtypeStruct((N_NODES, D_FEAT), jnp.float32),
    )(h0, b0.reshape(1, D_FEAT), W1, b1.reshape(1, D_FEAT), coef_col)
    return out


# speedup vs baseline: 122.6094x; 1.2083x over previous
import functools

import jax
import jax.numpy as jnp
from jax import lax
from jax.experimental import pallas as pl
from jax.experimental.pallas import tpu as pltpu
from jax.experimental.pallas import tpu_sc as plsc

N_NODES = 10000
D_FEAT = 128
N_EDGES = 320000

NS = 16
LANES = 16
E_TILE = N_EDGES // NS
E_COVER = 20096
NP = 10240
STRIPE = NP // NS
U = 10


def _edge_body(edge_hbm, coef_hbm,
               edges_v, cnt_v, t_v, dinv_loc, tmp_v, acc_v, dinv_v,
               stage_sh, dinv_sh, sem):
    s = lax.axis_index("s")
    ebase = s * E_TILE
    ebase_al = (ebase // 128) * 128
    boff = ebase - ebase_al
    nbase = s * STRIPE

    pltpu.sync_copy(edge_hbm.at[:, pl.ds(ebase_al, E_COVER)], edges_v)

    zeros16 = jnp.zeros((LANES,), jnp.float32)

    def zero_body(i, _):
        for u in range(8):
            off = i * 8 * LANES + u * LANES
            cnt_v[pl.ds(off, LANES)] = zeros16
            t_v[pl.ds(off, LANES)] = zeros16
        return _
    lax.fori_loop(0, NP // (8 * LANES), zero_body, None)

    ones16 = jnp.ones((LANES,), jnp.float32)

    def cnt_body(i, _):
        for u in range(U):
            off = boff + i * U * LANES + u * LANES
            sv = edges_v[0, pl.ds(off, LANES)]
            dv = edges_v[1, pl.ds(off, LANES)]
            plsc.addupdate_scatter(cnt_v, [dv], ones16, mask=sv != dv)
        return _
    lax.fori_loop(0, E_TILE // (U * LANES), cnt_body, None)

    pltpu.sync_copy(cnt_v, stage_sh.at[s])
    plsc.subcore_barrier()

    pltpu.sync_copy(stage_sh.at[0, pl.ds(nbase, STRIPE)], acc_v)
    for r in range(1, NS):
        pltpu.sync_copy(stage_sh.at[r, pl.ds(nbase, STRIPE)], tmp_v)

        def add_body(i, _, _r=r):
            for u in range(8):
                off = i * 8 * LANES + u * LANES
                acc_v[pl.ds(off, LANES)] = (
                    acc_v[pl.ds(off, LANES)] + tmp_v[pl.ds(off, LANES)])
            return _
        lax.fori_loop(0, STRIPE // (8 * LANES), add_body, None)

    def dinv_body(i, _):
        for u in range(8):
            off = i * 8 * LANES + u * LANES
            xdeg = acc_v[pl.ds(off, LANES)] + jnp.float32(1.0)
            ii = lax.bitcast_convert_type(xdeg, jnp.int32)
            ii = jnp.int32(0x5F3759DF) - (ii >> 1)
            y = lax.bitcast_convert_type(ii, jnp.float32)
            for _unused in range(3):
                y = y * (jnp.float32(1.5) - jnp.float32(0.5) * xdeg * y * y)
            dinv_v[pl.ds(off, LANES)] = y
        return _
    lax.fori_loop(0, STRIPE // (8 * LANES), dinv_body, None)
    pltpu.sync_copy(dinv_v, dinv_sh.at[pl.ds(nbase, STRIPE)])
    plsc.subcore_barrier()
    pltpu.sync_copy(dinv_sh, dinv_loc)

    def t_body(i, _):
        for u in range(U):
            off = boff + i * U * LANES + u * LANES
            sv = edges_v[0, pl.ds(off, LANES)]
            dv = edges_v[1, pl.ds(off, LANES)]
            g = plsc.load_gather(dinv_loc, [dv])
            plsc.addupdate_scatter(t_v, [sv], g, mask=sv != dv)
        return _
    lax.fori_loop(0, E_TILE // (U * LANES), t_body, None)

    pltpu.sync_copy(t_v, stage_sh.at[s])
    plsc.subcore_barrier()

    pltpu.sync_copy(stage_sh.at[0, pl.ds(nbase, STRIPE)], acc_v)
    for r in range(1, NS):
        pltpu.sync_copy(stage_sh.at[r, pl.ds(nbase, STRIPE)], tmp_v)

        def add2_body(i, _, _r=r):
            for u in range(8):
                off = i * 8 * LANES + u * LANES
                acc_v[pl.ds(off, LANES)] = (
                    acc_v[pl.ds(off, LANES)] + tmp_v[pl.ds(off, LANES)])
            return _
        lax.fori_loop(0, STRIPE // (8 * LANES), add2_body, None)

    def coef_body(i, _):
        for u in range(8):
            off = i * 8 * LANES + u * LANES
            dv = dinv_v[pl.ds(off, LANES)]
            acc_v[pl.ds(off, LANES)] = dv * (acc_v[pl.ds(off, LANES)] + dv)
        return _
    lax.fori_loop(0, STRIPE // (8 * LANES), coef_body, None)
    pltpu.sync_copy(acc_v, coef_hbm.at[pl.ds(nbase, STRIPE)])


_edge_kernel = pl.kernel(
    _edge_body,
    out_type=jax.ShapeDtypeStruct((NP,), jnp.float32),
    mesh=plsc.VectorSubcoreMesh(
        core_axis_name="c", subcore_axis_name="s", num_cores=1),
    compiler_params=pltpu.CompilerParams(needs_layout_passes=False),
    scratch_types=[
        pltpu.VMEM((2, E_COVER), jnp.int32),
        pltpu.VMEM((NP,), jnp.float32),
        pltpu.VMEM((NP,), jnp.float32),
        pltpu.VMEM((NP,), jnp.float32),
        pltpu.VMEM((STRIPE,), jnp.float32),
        pltpu.VMEM((STRIPE,), jnp.float32),
        pltpu.VMEM((STRIPE,), jnp.float32),
        pltpu.VMEM_SHARED((NS, NP), jnp.float32),
        pltpu.VMEM_SHARED((NP,), jnp.float32),
        pltpu.SemaphoreType.DMA,
    ],
)


def _dense_body(x_ref, w0_ref, b0c_ref, w1_ref, b1_ref, coef_ref, out_ref):
    coef = coef_ref[...]
    h1t = lax.dot_general(
        w0_ref[...], x_ref[...], (((1,), (1,)), ((), ())),
        preferred_element_type=jnp.float32)
    h1t = h1t * coef + b0c_ref[...]
    mean = jnp.mean(h1t, axis=1, keepdims=True)
    cen = h1t - mean
    var = jnp.mean(cen * cen, axis=1, keepdims=True)
    hn = cen * lax.rsqrt(var + jnp.float32(1e-5))
    hrt = jnp.maximum(hn, jnp.float32(0.0)) * coef
    out_ref[...] = lax.dot_general(
        hrt, w1_ref[...], (((0,), (1,)), ((), ())),
        preferred_element_type=jnp.float32) + b1_ref[...]


@functools.partial(jax.jit, static_argnames=())
def kernel(x, edge_index, W0, b0, W1, b1):
    coef_full = _edge_kernel(edge_index)
    coef_row = coef_full[:N_NODES].reshape(1, N_NODES)

    out = pl.pallas_call(
        _dense_body,
        out_shape=jax.ShapeDtypeStruct((N_NODES, D_FEAT), jnp.float32),
    )(x, W0, b0.reshape(D_FEAT, 1), W1, b1.reshape(1, D_FEAT), coef_row)
    return out


# speedup vs baseline: 135.0492x; 1.1015x over previous
import functools

import jax
import jax.numpy as jnp
from jax import lax
from jax.experimental import pallas as pl
from jax.experimental.pallas import tpu as pltpu
from jax.experimental.pallas import tpu_sc as plsc

N_NODES = 10000
D_FEAT = 128
N_EDGES = 320000

NS = 16
NC = 2
LANES = 16
E_TILE = N_EDGES // NS
E_HALF = E_TILE // NC
E_COVER = 20096
NP = 10240
STRIPE = NP // NS
U = 10


def _edge_body(edge_hbm, dinv_hbm, tpart_hbm,
               edges_v, cnt_v, t_v, dinv_loc, tmp_v, acc_v, dinv_v,
               stage_sh, dinv_sh, sem):
    c = lax.axis_index("c")
    s = lax.axis_index("s")
    ebase = s * E_TILE
    ebase_al = (ebase // 128) * 128
    boff = ebase - ebase_al
    nbase = s * STRIPE

    pltpu.sync_copy(edge_hbm.at[:, pl.ds(ebase_al, E_COVER)], edges_v)

    zeros16 = jnp.zeros((LANES,), jnp.float32)

    def zero_body(i, _):
        for u in range(8):
            off = i * 8 * LANES + u * LANES
            cnt_v[pl.ds(off, LANES)] = zeros16
            t_v[pl.ds(off, LANES)] = zeros16
        return _
    lax.fori_loop(0, NP // (8 * LANES), zero_body, None)

    ones16 = jnp.ones((LANES,), jnp.float32)

    def cnt_body(i, _):
        for u in range(U):
            off = boff + i * U * LANES + u * LANES
            sv = edges_v[0, pl.ds(off, LANES)]
            dv = edges_v[1, pl.ds(off, LANES)]
            plsc.addupdate_scatter(cnt_v, [dv], ones16, mask=sv != dv)
        return _
    lax.fori_loop(0, E_TILE // (U * LANES), cnt_body, None)

    pltpu.sync_copy(cnt_v, stage_sh.at[s])
    plsc.subcore_barrier()

    pltpu.sync_copy(stage_sh.at[0, pl.ds(nbase, STRIPE)], acc_v)
    for r in range(1, NS):
        pltpu.sync_copy(stage_sh.at[r, pl.ds(nbase, STRIPE)], tmp_v)

        def add_body(i, _, _r=r):
            for u in range(8):
                off = i * 8 * LANES + u * LANES
                acc_v[pl.ds(off, LANES)] = (
                    acc_v[pl.ds(off, LANES)] + tmp_v[pl.ds(off, LANES)])
            return _
        lax.fori_loop(0, STRIPE // (8 * LANES), add_body, None)

    def dinv_body(i, _):
        for u in range(8):
            off = i * 8 * LANES + u * LANES
            xdeg = acc_v[pl.ds(off, LANES)] + jnp.float32(1.0)
            ii = lax.bitcast_convert_type(xdeg, jnp.int32)
            ii = jnp.int32(0x5F3759DF) - (ii >> 1)
            y = lax.bitcast_convert_type(ii, jnp.float32)
            for _unused in range(3):
                y = y * (jnp.float32(1.5) - jnp.float32(0.5) * xdeg * y * y)
            dinv_v[pl.ds(off, LANES)] = y
        return _
    lax.fori_loop(0, STRIPE // (8 * LANES), dinv_body, None)
    pltpu.sync_copy(dinv_v, dinv_sh.at[pl.ds(nbase, STRIPE)])

    @pl.when(c == 0)
    def _write_dinv():
        pltpu.sync_copy(dinv_v, dinv_hbm.at[pl.ds(nbase, STRIPE)])

    plsc.subcore_barrier()
    pltpu.sync_copy(dinv_sh, dinv_loc)

    hoff = boff + c * E_HALF

    UT = 5
    def t_body(i, _):
        for u in range(UT):
            off = hoff + i * UT * LANES + u * LANES
            sv = edges_v[0, pl.ds(off, LANES)]
            dv = edges_v[1, pl.ds(off, LANES)]
            g = plsc.load_gather(dinv_loc, [dv])
            plsc.addupdate_scatter(t_v, [sv], g, mask=sv != dv)
        return _
    lax.fori_loop(0, E_HALF // (UT * LANES), t_body, None)

    pltpu.sync_copy(t_v, stage_sh.at[s])
    plsc.subcore_barrier()

    pltpu.sync_copy(stage_sh.at[0, pl.ds(nbase, STRIPE)], acc_v)
    for r in range(1, NS):
        pltpu.sync_copy(stage_sh.at[r, pl.ds(nbase, STRIPE)], tmp_v)

        def add2_body(i, _, _r=r):
            for u in range(8):
                off = i * 8 * LANES + u * LANES
                acc_v[pl.ds(off, LANES)] = (
                    acc_v[pl.ds(off, LANES)] + tmp_v[pl.ds(off, LANES)])
            return _
        lax.fori_loop(0, STRIPE // (8 * LANES), add2_body, None)

    pltpu.sync_copy(acc_v, tpart_hbm.at[pl.ds(c * NP + nbase, STRIPE)])


_edge_kernel = pl.kernel(
    _edge_body,
    out_type=(
        jax.ShapeDtypeStruct((NP,), jnp.float32),
        jax.ShapeDtypeStruct((NC * NP,), jnp.float32),
    ),
    mesh=plsc.VectorSubcoreMesh(
        core_axis_name="c", subcore_axis_name="s", num_cores=NC),
    compiler_params=pltpu.CompilerParams(needs_layout_passes=False),
    scratch_types=[
        pltpu.VMEM((2, E_COVER), jnp.int32),
        pltpu.VMEM((NP,), jnp.float32),
        pltpu.VMEM((NP,), jnp.float32),
        pltpu.VMEM((NP,), jnp.float32),
        pltpu.VMEM((STRIPE,), jnp.float32),
        pltpu.VMEM((STRIPE,), jnp.float32),
        pltpu.VMEM((STRIPE,), jnp.float32),
        pltpu.VMEM_SHARED((NS, NP), jnp.float32),
        pltpu.VMEM_SHARED((NP,), jnp.float32),
        pltpu.SemaphoreType.DMA,
    ],
)


def _mm0_body(w0_ref, x_ref, xw_ref):
    xw_ref[...] = lax.dot_general(
        w0_ref[...], x_ref[...], (((1,), (1,)), ((), ())),
        preferred_element_type=jnp.float32)


def _dense_body(xw_ref, b0c_ref, w1_ref, b1_ref, dinv_ref, tp_ref, out_ref):
    dv = dinv_ref[0:1, 0:N_NODES]
    t = tp_ref[0:1, 0:N_NODES] + tp_ref[0:1, NP:NP + N_NODES]
    coef = dv * (t + dv)
    h1t = xw_ref[...] * coef + b0c_ref[...]
    mean = jnp.mean(h1t, axis=1, keepdims=True)
    cen = h1t - mean
    var = jnp.mean(cen * cen, axis=1, keepdims=True)
    hn = cen * lax.rsqrt(var + jnp.float32(1e-5))
    hrt = jnp.maximum(hn, jnp.float32(0.0)) * coef
    out_ref[...] = lax.dot_general(
        hrt, w1_ref[...], (((0,), (1,)), ((), ())),
        preferred_element_type=jnp.float32) + b1_ref[...]


@functools.partial(jax.jit, static_argnames=())
def kernel(x, edge_index, W0, b0, W1, b1):
    dinv_full, t_part = _edge_kernel(edge_index)

    xw0t = pl.pallas_call(
        _mm0_body,
        out_shape=jax.ShapeDtypeStruct((D_FEAT, N_NODES), jnp.float32),
    )(W0, x)

    out = pl.pallas_call(
        _dense_body,
        out_shape=jax.ShapeDtypeStruct((N_NODES, D_FEAT), jnp.float32),
    )(xw0t, b0.reshape(D_FEAT, 1), W1, b1.reshape(1, D_FEAT),
      dinv_full.reshape(1, NP), t_part.reshape(1, NC * NP))
    return out
